# baseline (speedup 1.0000x reference)
import functools

import jax
import jax.numpy as jnp
from jax import lax
from jax.experimental import pallas as pl
from jax.experimental.pallas import tpu as pltpu
from jax.experimental.pallas import tpu_sc as plsc

N_NODES = 10000
N_EDGES = 320000
D_IN = 128
D_H = 32
N_REL = 2

NC = 2
NS = 16
NTILES = NC * NS
CHUNK = 128
NBUF = 2
NCHUNK = 80
CH_C0 = 56
CH_C1 = 104
CH_MAX = 104
EPT = NCHUNK * CHUNK
E_PAD = EPT * NTILES
EROWS = N_EDGES // 128
EROWS_PAD = E_PAD // 128

TQ = 2 * N_NODES
SENT = TQ
TQP = TQ + 8
ACCW = 36


def _prep_body(src_ref, dst_ref, et_ref, gsrc_ref, gq_ref, dstp_ref):
    src = src_ref[...]
    dst = dst_ref[...]
    et = et_ref[...]
    rows = lax.broadcasted_iota(jnp.int32, (EROWS_PAD, 128), 0)
    valid = rows < EROWS
    gsrc_ref[...] = jnp.where(valid, et * N_NODES + src, 0)
    gq_ref[...] = jnp.where(valid, et * N_NODES + dst, SENT)
    dstp_ref[...] = jnp.where(valid, dst, 0)


_prep = pl.pallas_call(
    _prep_body,
    out_shape=(
        jax.ShapeDtypeStruct((EROWS_PAD, 128), jnp.int32),
        jax.ShapeDtypeStruct((EROWS_PAD, 128), jnp.int32),
        jax.ShapeDtypeStruct((EROWS_PAD, 128), jnp.int32),
    ),
)


def _project(h, W, q, k, xw_ref, qn_ref, kn_ref):
    for r in range(N_REL):
        xw = jnp.dot(h, W[r], preferred_element_type=jnp.float32)
        xw_ref[r] = xw
        qn_ref[r : r + 1, :] = lax.dot_general(
            q, xw, (((0,), (1,)), ((), ())), preferred_element_type=jnp.float32)
        kn_ref[r : r + 1, :] = lax.dot_general(
            k, xw, (((0,), (1,)), ((), ())), preferred_element_type=jnp.float32)


def _proj0_body(x_ref, W_ref, q_ref, k_ref, xw_ref, qn_ref, kn_ref):
    _project(x_ref[...], W_ref[...], q_ref[...], k_ref[...],
             xw_ref, qn_ref, kn_ref)


_proj0 = pl.pallas_call(
    _proj0_body,
    out_shape=(
        jax.ShapeDtypeStruct((N_REL, N_NODES, D_H), jnp.float32),
        jax.ShapeDtypeStruct((N_REL, N_NODES), jnp.float32),
        jax.ShapeDtypeStruct((N_REL, N_NODES), jnp.float32),
    ),
)


def _norm_h(acc_ref, den_ref, b_ref, g_ref, be_ref):
    sacc = acc_ref[0] + acc_ref[1]
    ones = jnp.ones((NTILES, 1), jnp.float32)
    den = lax.dot_general(den_ref[...], ones, (((0,), (0,)), ((), ())),
                          preferred_element_type=jnp.float32)
    h = sacc / (den + 1e-16) + b_ref[...]
    h = jnp.maximum(h, 0.0)
    mu = jnp.mean(h, axis=0, keepdims=True)
    var = jnp.mean((h - mu) * (h - mu), axis=0, keepdims=True)
    return (h - mu) * lax.rsqrt(var + 1e-5) * g_ref[...] + be_ref[...]


def _finproj_body(acc_ref, den_ref, b_ref, g_ref, be_ref, W_ref, q_ref, k_ref,
                  xw_ref, qn_ref, kn_ref):
    hn = _norm_h(acc_ref, den_ref, b_ref, g_ref, be_ref)
    _project(hn, W_ref[...], q_ref[...], k_ref[...], xw_ref, qn_ref, kn_ref)


_finproj = pl.pallas_call(
    _finproj_body,
    out_shape=(
        jax.ShapeDtypeStruct((N_REL, N_NODES, D_H), jnp.float32),
        jax.ShapeDtypeStruct((N_REL, N_NODES), jnp.float32),
        jax.ShapeDtypeStruct((N_REL, N_NODES), jnp.float32),
    ),
)


def _head_body(acc_ref, den_ref, b_ref, g_ref, be_ref, mW1_ref, mb1_ref,
               mW2_ref, mb2_ref, out_ref):
    hn = _norm_h(acc_ref, den_ref, b_ref, g_ref, be_ref)
    z = jnp.dot(hn, mW1_ref[...], preferred_element_type=jnp.float32)
    hh = jax.nn.sigmoid(z + mb1_ref[...])
    out_ref[...] = (jnp.dot(hh, mW2_ref[...], preferred_element_type=jnp.float32)
                    + mb2_ref[...])


_head = pl.pallas_call(
    _head_body,
    out_shape=jax.ShapeDtypeStruct((N_NODES, 2), jnp.float32),
)


_mesh = plsc.VectorSubcoreMesh(core_axis_name="c", subcore_axis_name="s")


@functools.partial(
    pl.kernel,
    out_type=(
        jax.ShapeDtypeStruct((NC * N_NODES, D_H), jnp.float32),
        jax.ShapeDtypeStruct((NTILES, N_NODES), jnp.float32),
    ),
    mesh=_mesh,
    scratch_types=[
        pltpu.VMEM((TQP,), jnp.float32),
        pltpu.VMEM((TQP,), jnp.float32),
        pltpu.VMEM((CH_MAX, 128), jnp.int32),
        pltpu.VMEM((CH_MAX, 128), jnp.int32),
        pltpu.VMEM((CH_MAX, 128), jnp.int32),
        pltpu.VMEM((CHUNK + 16,), jnp.float32),
        pltpu.VMEM((N_NODES,), jnp.float32),
    ]
    + [pltpu.VMEM((CHUNK,), jnp.int32) for _ in range(NBUF)]
    + [pltpu.VMEM((CHUNK, D_H), jnp.float32) for _ in range(NBUF)]
    + [pltpu.VMEM((CHUNK, D_H), jnp.float32) for _ in range(NBUF)]
    + [pltpu.SemaphoreType.DMA for _ in range(3 * NBUF)]
    + [pltpu.VMEM_SHARED((N_NODES, D_H), jnp.float32)],
    compiler_params=pltpu.CompilerParams(needs_layout_passes=False,
                                         use_tc_tiling_on_sc=False),
)
def _edge_kernel(xwf, qnf, knf, gsrcH, gqH, dstH, out, den_out,
                 qn_t, kn_t, eb_src, eb_gq, eb_dst, ex_v, den_t,
                 d0, d1, r0, r1, p0, p1,
                 sg0, sg1, sh0, sh1, ss0, ss1, acc_s):
    dst_v = [d0, d1]
    rows = [r0, r1]
    scb = [p0, p1]
    sg = [sg0, sg1]
    sh = [sh0, sh1]
    ss = [ss0, ss1]
    c = lax.axis_index("c")
    s = lax.axis_index("s")
    tid = c * NS + s

    pltpu.sync_copy(qnf, qn_t)
    pltpu.sync_copy(knf, kn_t)
    base = jnp.where(c == 0, s * CH_C0, NS * CH_C0 + s * CH_C1)
    pltpu.sync_copy(gsrcH.at[pl.ds(base, CH_MAX)], eb_src)
    pltpu.sync_copy(gqH.at[pl.ds(base, CH_MAX)], eb_gq)
    pltpu.sync_copy(dstH.at[pl.ds(base, CH_MAX)], eb_dst)

    zero16 = jnp.zeros((16,), jnp.float32)

    def zden_body(j, _):
        den_t[pl.ds(j * 16, 16)] = zero16
        return 0

    lax.fori_loop(0, N_NODES // 16, zden_body, 0)
    ex_v[pl.ds(CHUNK, 16)] = zero16

    def zbuf_body(j, _):
        scb[0][j, 0:16] = zero16
        scb[0][j, 16:32] = zero16
        return 0

    lax.fori_loop(0, CHUNK, zbuf_body, 0)

    nacc_full = N_NODES // 128
    nacc_rem = N_NODES - nacc_full * 128

    def zacc_body(i, _):
        j = s + i * NS

        @pl.when(j < nacc_full)
        def _():
            pltpu.sync_copy(scb[0], acc_s.at[pl.ds(j * 128, 128)])

        @pl.when(j == nacc_full)
        def _():
            pltpu.sync_copy(scb[0].at[pl.ds(0, nacc_rem)],
                            acc_s.at[pl.ds(nacc_full * 128, nacc_rem)])

        return 0

    lax.fori_loop(0, nacc_full // NS + 1, zacc_body, 0)
    plsc.subcore_barrier()

    for b in range(NBUF):
        pltpu.async_copy(xwf.at[eb_src.at[b, pl.ds(0, 64)]],
                         rows[b].at[pl.ds(0, 64)], sg[b])
        pltpu.async_copy(xwf.at[eb_src.at[b, pl.ds(64, 64)]],
                         rows[b].at[pl.ds(64, 64)], sh[b])

    ngroup = jnp.where(c == 0, CH_C0 // NBUF, CH_C1 // NBUF)

    def group_body(i, _):
        for b in range(NBUF):
            ch = i * NBUF + b
            pltpu.make_async_copy(xwf.at[eb_src.at[b, pl.ds(0, 64)]],
                                  rows[b].at[pl.ds(0, 64)], sg[b]).wait()
            pltpu.make_async_copy(xwf.at[eb_src.at[b, pl.ds(64, 64)]],
                                  rows[b].at[pl.ds(64, 64)], sh[b]).wait()

            @pl.when(i > 0)
            def _():
                pltpu.make_async_copy(scb[b], acc_s.at[dst_v[b]],
                                      ss[b]).wait()

            def icopy_body(k2, _, ch=ch, b=b):
                dst_v[b][pl.ds(k2 * 16, 16)] = eb_dst[ch, pl.ds(k2 * 16, 16)]
                return 0

            lax.fori_loop(0, CHUNK // 16, icopy_body, 0)

            def alpha_body(k2, _, ch=ch, b=b):
                idxq = eb_gq[ch, pl.ds(k2 * 16, 16)]
                idxs = eb_src[ch, pl.ds(k2 * 16, 16)]
                idxd = dst_v[b][pl.ds(k2 * 16, 16)]
                t = (plsc.load_gather(qn_t, [idxq])
                     + plsc.load_gather(kn_t, [idxs]))
                ex = jnp.exp(jnp.maximum(t, 0.2 * t))
                ex_v[pl.ds(k2 * 16, 16)] = ex
                plsc.addupdate_scatter(den_t, [idxd], ex)
                return 0

            lax.fori_loop(0, CHUNK // 16, alpha_body, 0)

            @plsc.parallel_loop(0, CHUNK, 1, unroll=8)
            def scale_body(e, b=b):
                exs = ex_v[pl.ds(e, 16)][0]
                scb[b][e, 0:16] = rows[b][e, 0:16] * exs
                scb[b][e, 16:32] = rows[b][e, 16:32] * exs

            @pl.when(i < ngroup - 1)
            def _():
                pltpu.async_copy(xwf.at[eb_src.at[ch + NBUF, pl.ds(0, 64)]],
                                 rows[b].at[pl.ds(0, 64)], sg[b])
                pltpu.async_copy(xwf.at[eb_src.at[ch + NBUF, pl.ds(64, 64)]],
                                 rows[b].at[pl.ds(64, 64)], sh[b])

            pltpu.async_copy(scb[b], acc_s.at[dst_v[b]], ss[b], add=True)
        return 0

    lax.fori_loop(0, ngroup, group_body, 0)

    for b in range(NBUF):
        pltpu.make_async_copy(scb[b], acc_s.at[dst_v[b]], ss[b]).wait()

    pltpu.sync_copy(den_t, den_out.at[tid])
    plsc.subcore_barrier()

    @pl.when(s == 0)
    def _():
        pltpu.sync_copy(acc_s, out.at[pl.ds(c * N_NODES, N_NODES)])


def kernel(x, edge_index, edge_type, W0, q0, k0, b0, g0, be0,
           W1, q1, k1, b1, g1, be1, W2, q2, k2, b2, g2, be2,
           mW1, mb1, mW2, mb2):
    f32 = jnp.float32
    zpad = jnp.zeros((EROWS_PAD - EROWS, 128), jnp.int32)
    src_p = jnp.concatenate([edge_index[0].reshape(EROWS, 128), zpad])
    dst_p = jnp.concatenate([edge_index[1].reshape(EROWS, 128), zpad])
    et_p = jnp.concatenate([edge_type.reshape(EROWS, 128), zpad])
    gsrc, gq, dstp = _prep(src_p, dst_p, et_p)

    def _tables(qn, kn):
        qnf = jnp.concatenate([qn.reshape(TQ), jnp.full((TQP - TQ,), -1e30, f32)])
        knf = jnp.concatenate([kn.reshape(TQ), jnp.zeros((TQP - TQ,), f32)])
        return qnf, knf

    def _edge(xw, qn, kn):
        qnf, knf = _tables(qn, kn)
        acc, den = _edge_kernel(xw.reshape(TQ, D_H), qnf, knf, gsrc, gq, dstp)
        return acc.reshape(NC, N_NODES, D_H), den

    xw, qn, kn = _proj0(x, W0, q0, k0)
    acc, den = _edge(xw, qn, kn)

    layer_params = [(b0, g0, be0, W1, q1, k1), (b1, g1, be1, W2, q2, k2)]
    for b, g, be, W, q, k in layer_params:
        xw, qn, kn = _finproj(acc, den, b.reshape(1, D_H), g.reshape(1, D_H),
                              be.reshape(1, D_H), W, q, k)
        acc, den = _edge(xw, qn, kn)

    return _head(acc, den, b2.reshape(1, D_H), g2.reshape(1, D_H),
                 be2.reshape(1, D_H), mW1, mb1.reshape(1, D_H), mW2,
                 mb2.reshape(1, 2))


# speedup vs baseline: 1.1440x; 1.1440x over previous
import functools

import jax
import jax.numpy as jnp
from jax import lax
from jax.experimental import pallas as pl
from jax.experimental.pallas import tpu as pltpu
from jax.experimental.pallas import tpu_sc as plsc

N_NODES = 10000
N_EDGES = 320000
D_IN = 128
D_H = 32
N_REL = 2

NC = 2
NS = 16
NTILES = NC * NS
CHUNK = 128
NBUF = 2
NCHUNK = 80
CH_C0 = 104
CH_C1 = 56
CH_MAX = 104
EPT = NCHUNK * CHUNK
E_PAD = EPT * NTILES
EROWS = N_EDGES // 128
EROWS_PAD = E_PAD // 128

TQ = 2 * N_NODES
SENT = TQ
TQP = TQ + 8
ACCW = 36


def _prep_body(src_ref, dst_ref, et_ref, gsrc_ref, gq_ref, dstp_ref):
    src = src_ref[...]
    dst = dst_ref[...]
    et = et_ref[...]
    rows = lax.broadcasted_iota(jnp.int32, (EROWS_PAD, 128), 0)
    valid = rows < EROWS
    gsrc_ref[...] = jnp.where(valid, et * N_NODES + src, 0)
    gq_ref[...] = jnp.where(valid, et * N_NODES + dst, SENT)
    dstp_ref[...] = jnp.where(valid, dst, 0)


_prep = pl.pallas_call(
    _prep_body,
    out_shape=(
        jax.ShapeDtypeStruct((EROWS_PAD, 128), jnp.int32),
        jax.ShapeDtypeStruct((EROWS_PAD, 128), jnp.int32),
        jax.ShapeDtypeStruct((EROWS_PAD, 128), jnp.int32),
    ),
)


def _project(h, W, q, k, xw_ref, qn_ref, kn_ref):
    for r in range(N_REL):
        xw = jnp.dot(h, W[r], preferred_element_type=jnp.float32)
        xw_ref[r] = xw
        qn_ref[r : r + 1, :] = lax.dot_general(
            q, xw, (((0,), (1,)), ((), ())), preferred_element_type=jnp.float32)
        kn_ref[r : r + 1, :] = lax.dot_general(
            k, xw, (((0,), (1,)), ((), ())), preferred_element_type=jnp.float32)


def _proj0_body(x_ref, W_ref, q_ref, k_ref, xw_ref, qn_ref, kn_ref):
    _project(x_ref[...], W_ref[...], q_ref[...], k_ref[...],
             xw_ref, qn_ref, kn_ref)


_proj0 = pl.pallas_call(
    _proj0_body,
    out_shape=(
        jax.ShapeDtypeStruct((N_REL, N_NODES, D_H), jnp.float32),
        jax.ShapeDtypeStruct((N_REL, N_NODES), jnp.float32),
        jax.ShapeDtypeStruct((N_REL, N_NODES), jnp.float32),
    ),
)


def _norm_h(acc_ref, den_ref, b_ref, g_ref, be_ref):
    sacc = acc_ref[0] + acc_ref[1]
    ones = jnp.ones((NTILES, 1), jnp.float32)
    den = lax.dot_general(den_ref[...], ones, (((0,), (0,)), ((), ())),
                          preferred_element_type=jnp.float32)
    h = sacc / (den + 1e-16) + b_ref[...]
    h = jnp.maximum(h, 0.0)
    mu = jnp.mean(h, axis=0, keepdims=True)
    var = jnp.mean((h - mu) * (h - mu), axis=0, keepdims=True)
    return (h - mu) * lax.rsqrt(var + 1e-5) * g_ref[...] + be_ref[...]


def _finproj_body(acc_ref, den_ref, b_ref, g_ref, be_ref, W_ref, q_ref, k_ref,
                  xw_ref, qn_ref, kn_ref):
    hn = _norm_h(acc_ref, den_ref, b_ref, g_ref, be_ref)
    _project(hn, W_ref[...], q_ref[...], k_ref[...], xw_ref, qn_ref, kn_ref)


_finproj = pl.pallas_call(
    _finproj_body,
    out_shape=(
        jax.ShapeDtypeStruct((N_REL, N_NODES, D_H), jnp.float32),
        jax.ShapeDtypeStruct((N_REL, N_NODES), jnp.float32),
        jax.ShapeDtypeStruct((N_REL, N_NODES), jnp.float32),
    ),
)


def _head_body(acc_ref, den_ref, b_ref, g_ref, be_ref, mW1_ref, mb1_ref,
               mW2_ref, mb2_ref, out_ref):
    hn = _norm_h(acc_ref, den_ref, b_ref, g_ref, be_ref)
    z = jnp.dot(hn, mW1_ref[...], preferred_element_type=jnp.float32)
    hh = jax.nn.sigmoid(z + mb1_ref[...])
    out_ref[...] = (jnp.dot(hh, mW2_ref[...], preferred_element_type=jnp.float32)
                    + mb2_ref[...])


_head = pl.pallas_call(
    _head_body,
    out_shape=jax.ShapeDtypeStruct((N_NODES, 2), jnp.float32),
)


_mesh = plsc.VectorSubcoreMesh(core_axis_name="c", subcore_axis_name="s")


@functools.partial(
    pl.kernel,
    out_type=(
        jax.ShapeDtypeStruct((NC * N_NODES, D_H), jnp.float32),
        jax.ShapeDtypeStruct((NTILES, N_NODES), jnp.float32),
    ),
    mesh=_mesh,
    scratch_types=[
        pltpu.VMEM((TQP,), jnp.float32),
        pltpu.VMEM((TQP,), jnp.float32),
        pltpu.VMEM((CH_MAX, 128), jnp.int32),
        pltpu.VMEM((CH_MAX, 128), jnp.int32),
        pltpu.VMEM((CH_MAX, 128), jnp.int32),
        pltpu.VMEM((CHUNK + 16,), jnp.float32),
        pltpu.VMEM((N_NODES,), jnp.float32),
    ]
    + [pltpu.VMEM((CHUNK,), jnp.int32) for _ in range(NBUF)]
    + [pltpu.VMEM((CHUNK, D_H), jnp.float32) for _ in range(NBUF)]
    + [pltpu.VMEM((CHUNK, D_H), jnp.float32) for _ in range(NBUF)]
    + [pltpu.SemaphoreType.DMA for _ in range(3 * NBUF)]
    + [pltpu.VMEM_SHARED((N_NODES, D_H), jnp.float32)],
    compiler_params=pltpu.CompilerParams(needs_layout_passes=False,
                                         use_tc_tiling_on_sc=False),
)
def _edge_kernel(xwf, qnf, knf, gsrcH, gqH, dstH, out, den_out,
                 qn_t, kn_t, eb_src, eb_gq, eb_dst, ex_v, den_t,
                 d0, d1, r0, r1, p0, p1,
                 sg0, sg1, sh0, sh1, ss0, ss1, acc_s):
    dst_v = [d0, d1]
    rows = [r0, r1]
    scb = [p0, p1]
    sg = [sg0, sg1]
    sh = [sh0, sh1]
    ss = [ss0, ss1]
    c = lax.axis_index("c")
    s = lax.axis_index("s")
    tid = c * NS + s

    pltpu.sync_copy(qnf, qn_t)
    pltpu.sync_copy(knf, kn_t)
    base = jnp.where(c == 0, s * CH_C0, NS * CH_C0 + s * CH_C1)
    pltpu.sync_copy(gsrcH.at[pl.ds(base, CH_MAX)], eb_src)
    pltpu.sync_copy(gqH.at[pl.ds(base, CH_MAX)], eb_gq)
    pltpu.sync_copy(dstH.at[pl.ds(base, CH_MAX)], eb_dst)

    zero16 = jnp.zeros((16,), jnp.float32)

    def zden_body(j, _):
        den_t[pl.ds(j * 16, 16)] = zero16
        return 0

    lax.fori_loop(0, N_NODES // 16, zden_body, 0)
    ex_v[pl.ds(CHUNK, 16)] = zero16

    def zbuf_body(j, _):
        scb[0][j, 0:16] = zero16
        scb[0][j, 16:32] = zero16
        return 0

    lax.fori_loop(0, CHUNK, zbuf_body, 0)

    nacc_full = N_NODES // 128
    nacc_rem = N_NODES - nacc_full * 128

    def zacc_body(i, _):
        j = s + i * NS

        @pl.when(j < nacc_full)
        def _():
            pltpu.sync_copy(scb[0], acc_s.at[pl.ds(j * 128, 128)])

        @pl.when(j == nacc_full)
        def _():
            pltpu.sync_copy(scb[0].at[pl.ds(0, nacc_rem)],
                            acc_s.at[pl.ds(nacc_full * 128, nacc_rem)])

        return 0

    lax.fori_loop(0, nacc_full // NS + 1, zacc_body, 0)
    plsc.subcore_barrier()

    for b in range(NBUF):
        pltpu.async_copy(xwf.at[eb_src.at[b, pl.ds(0, 64)]],
                         rows[b].at[pl.ds(0, 64)], sg[b])
        pltpu.async_copy(xwf.at[eb_src.at[b, pl.ds(64, 64)]],
                         rows[b].at[pl.ds(64, 64)], sh[b])

    ngroup = jnp.where(c == 0, CH_C0 // NBUF, CH_C1 // NBUF)

    def group_body(i, _):
        for b in range(NBUF):
            ch = i * NBUF + b
            pltpu.make_async_copy(xwf.at[eb_src.at[b, pl.ds(0, 64)]],
                                  rows[b].at[pl.ds(0, 64)], sg[b]).wait()
            pltpu.make_async_copy(xwf.at[eb_src.at[b, pl.ds(64, 64)]],
                                  rows[b].at[pl.ds(64, 64)], sh[b]).wait()

            @pl.when(i > 0)
            def _():
                pltpu.make_async_copy(scb[b], acc_s.at[dst_v[b]],
                                      ss[b]).wait()

            def icopy_body(k2, _, ch=ch, b=b):
                dst_v[b][pl.ds(k2 * 16, 16)] = eb_dst[ch, pl.ds(k2 * 16, 16)]
                return 0

            lax.fori_loop(0, CHUNK // 16, icopy_body, 0)

            def alpha_body(k2, _, ch=ch, b=b):
                idxq = eb_gq[ch, pl.ds(k2 * 16, 16)]
                idxs = eb_src[ch, pl.ds(k2 * 16, 16)]
                idxd = dst_v[b][pl.ds(k2 * 16, 16)]
                t = (plsc.load_gather(qn_t, [idxq])
                     + plsc.load_gather(kn_t, [idxs]))
                ex = jnp.exp(jnp.maximum(t, 0.2 * t))
                ex_v[pl.ds(k2 * 16, 16)] = ex
                plsc.addupdate_scatter(den_t, [idxd], ex)
                return 0

            lax.fori_loop(0, CHUNK // 16, alpha_body, 0)

            @plsc.parallel_loop(0, CHUNK, 1, unroll=8)
            def scale_body(e, b=b):
                exs = ex_v[pl.ds(e, 16)][0]
                scb[b][e, 0:16] = rows[b][e, 0:16] * exs
                scb[b][e, 16:32] = rows[b][e, 16:32] * exs

            @pl.when(i < ngroup - 1)
            def _():
                pltpu.async_copy(xwf.at[eb_src.at[ch + NBUF, pl.ds(0, 64)]],
                                 rows[b].at[pl.ds(0, 64)], sg[b])
                pltpu.async_copy(xwf.at[eb_src.at[ch + NBUF, pl.ds(64, 64)]],
                                 rows[b].at[pl.ds(64, 64)], sh[b])

            pltpu.async_copy(scb[b], acc_s.at[dst_v[b]], ss[b], add=True)
        return 0

    lax.fori_loop(0, ngroup, group_body, 0)

    for b in range(NBUF):
        pltpu.make_async_copy(scb[b], acc_s.at[dst_v[b]], ss[b]).wait()

    pltpu.sync_copy(den_t, den_out.at[tid])
    plsc.subcore_barrier()

    @pl.when(s == 0)
    def _():
        pltpu.sync_copy(acc_s, out.at[pl.ds(c * N_NODES, N_NODES)])


def kernel(x, edge_index, edge_type, W0, q0, k0, b0, g0, be0,
           W1, q1, k1, b1, g1, be1, W2, q2, k2, b2, g2, be2,
           mW1, mb1, mW2, mb2):
    f32 = jnp.float32
    zpad = jnp.zeros((EROWS_PAD - EROWS, 128), jnp.int32)
    src_p = jnp.concatenate([edge_index[0].reshape(EROWS, 128), zpad])
    dst_p = jnp.concatenate([edge_index[1].reshape(EROWS, 128), zpad])
    et_p = jnp.concatenate([edge_type.reshape(EROWS, 128), zpad])
    gsrc, gq, dstp = _prep(src_p, dst_p, et_p)

    def _tables(qn, kn):
        qnf = jnp.concatenate([qn.reshape(TQ), jnp.full((TQP - TQ,), -1e30, f32)])
        knf = jnp.concatenate([kn.reshape(TQ), jnp.zeros((TQP - TQ,), f32)])
        return qnf, knf

    def _edge(xw, qn, kn):
        qnf, knf = _tables(qn, kn)
        acc, den = _edge_kernel(xw.reshape(TQ, D_H), qnf, knf, gsrc, gq, dstp)
        return acc.reshape(NC, N_NODES, D_H), den

    xw, qn, kn = _proj0(x, W0, q0, k0)
    acc, den = _edge(xw, qn, kn)

    layer_params = [(b0, g0, be0, W1, q1, k1), (b1, g1, be1, W2, q2, k2)]
    for b, g, be, W, q, k in layer_params:
        xw, qn, kn = _finproj(acc, den, b.reshape(1, D_H), g.reshape(1, D_H),
                              be.reshape(1, D_H), W, q, k)
        acc, den = _edge(xw, qn, kn)

    return _head(acc, den, b2.reshape(1, D_H), g2.reshape(1, D_H),
                 be2.reshape(1, D_H), mW1, mb1.reshape(1, D_H), mW2,
                 mb2.reshape(1, 2))


# speedup vs baseline: 1.1574x; 1.0118x over previous
import functools

import jax
import jax.numpy as jnp
from jax import lax
from jax.experimental import pallas as pl
from jax.experimental.pallas import tpu as pltpu
from jax.experimental.pallas import tpu_sc as plsc

N_NODES = 10000
N_EDGES = 320000
D_IN = 128
D_H = 32
N_REL = 2

NC = 2
NS = 16
NTILES = NC * NS
CHUNK = 128
NBUF = 2
NCHUNK = 80
CH_C0 = 104
CH_C1 = 56
CH_MAX = 104
EPT = NCHUNK * CHUNK
E_PAD = EPT * NTILES
EROWS = N_EDGES // 128
EROWS_PAD = E_PAD // 128

TQ = 2 * N_NODES
SENT = TQ
TQP = TQ + 8
ACCW = 36


def _prep_body(src_ref, dst_ref, et_ref, gsrc_ref, gq_ref, dstp_ref):
    src = src_ref[...]
    dst = dst_ref[...]
    et = et_ref[...]
    rows = lax.broadcasted_iota(jnp.int32, (EROWS_PAD, 128), 0)
    valid = rows < EROWS
    gsrc_ref[...] = jnp.where(valid, et * N_NODES + src, 0)
    gq_ref[...] = jnp.where(valid, et * N_NODES + dst, SENT)
    dstp_ref[...] = jnp.where(valid, dst, 0)


_prep = pl.pallas_call(
    _prep_body,
    out_shape=(
        jax.ShapeDtypeStruct((EROWS_PAD, 128), jnp.int32),
        jax.ShapeDtypeStruct((EROWS_PAD, 128), jnp.int32),
        jax.ShapeDtypeStruct((EROWS_PAD, 128), jnp.int32),
    ),
)


def _project(h, W, q, k, xw_ref, qn_ref, kn_ref):
    for r in range(N_REL):
        xw = jnp.dot(h, W[r], preferred_element_type=jnp.float32)
        xw_ref[r] = xw
        qn_ref[r : r + 1, :] = lax.dot_general(
            q, xw, (((0,), (1,)), ((), ())), preferred_element_type=jnp.float32)
        kn_ref[r : r + 1, :] = lax.dot_general(
            k, xw, (((0,), (1,)), ((), ())), preferred_element_type=jnp.float32)


def _proj0_body(x_ref, W_ref, q_ref, k_ref, xw_ref, qn_ref, kn_ref):
    _project(x_ref[...], W_ref[...], q_ref[...], k_ref[...],
             xw_ref, qn_ref, kn_ref)


_proj0 = pl.pallas_call(
    _proj0_body,
    out_shape=(
        jax.ShapeDtypeStruct((N_REL, N_NODES, D_H), jnp.float32),
        jax.ShapeDtypeStruct((N_REL, N_NODES), jnp.float32),
        jax.ShapeDtypeStruct((N_REL, N_NODES), jnp.float32),
    ),
)


def _norm_h(acc_ref, den_ref, b_ref, g_ref, be_ref):
    sacc = acc_ref[0] + acc_ref[1]
    ones = jnp.ones((NTILES, 1), jnp.float32)
    den = lax.dot_general(den_ref[...], ones, (((0,), (0,)), ((), ())),
                          preferred_element_type=jnp.float32)
    h = sacc / (den + 1e-16) + b_ref[...]
    h = jnp.maximum(h, 0.0)
    mu = jnp.mean(h, axis=0, keepdims=True)
    var = jnp.mean((h - mu) * (h - mu), axis=0, keepdims=True)
    return (h - mu) * lax.rsqrt(var + 1e-5) * g_ref[...] + be_ref[...]


def _finproj_body(acc_ref, den_ref, b_ref, g_ref, be_ref, W_ref, q_ref, k_ref,
                  xw_ref, qn_ref, kn_ref):
    hn = _norm_h(acc_ref, den_ref, b_ref, g_ref, be_ref)
    _project(hn, W_ref[...], q_ref[...], k_ref[...], xw_ref, qn_ref, kn_ref)


_finproj = pl.pallas_call(
    _finproj_body,
    out_shape=(
        jax.ShapeDtypeStruct((N_REL, N_NODES, D_H), jnp.float32),
        jax.ShapeDtypeStruct((N_REL, N_NODES), jnp.float32),
        jax.ShapeDtypeStruct((N_REL, N_NODES), jnp.float32),
    ),
)


def _head_body(acc_ref, den_ref, b_ref, g_ref, be_ref, mW1_ref, mb1_ref,
               mW2_ref, mb2_ref, out_ref):
    hn = _norm_h(acc_ref, den_ref, b_ref, g_ref, be_ref)
    z = jnp.dot(hn, mW1_ref[...], preferred_element_type=jnp.float32)
    hh = jax.nn.sigmoid(z + mb1_ref[...])
    out_ref[...] = (jnp.dot(hh, mW2_ref[...], preferred_element_type=jnp.float32)
                    + mb2_ref[...])


_head = pl.pallas_call(
    _head_body,
    out_shape=jax.ShapeDtypeStruct((N_NODES, 2), jnp.float32),
)


_mesh = plsc.VectorSubcoreMesh(core_axis_name="c", subcore_axis_name="s")


@functools.partial(
    pl.kernel,
    out_type=(
        jax.ShapeDtypeStruct((NC * N_NODES, D_H), jnp.float32),
        jax.ShapeDtypeStruct((NTILES, N_NODES), jnp.float32),
    ),
    mesh=_mesh,
    scratch_types=[
        pltpu.VMEM((TQP,), jnp.float32),
        pltpu.VMEM((TQP,), jnp.float32),
        pltpu.VMEM((CH_MAX, 128), jnp.int32),
        pltpu.VMEM((CH_MAX, 128), jnp.int32),
        pltpu.VMEM((CH_MAX, 128), jnp.int32),
        pltpu.VMEM((CHUNK + 16,), jnp.float32),
        pltpu.VMEM((N_NODES,), jnp.float32),
    ]
    + [pltpu.VMEM((CHUNK,), jnp.int32) for _ in range(NBUF)]
    + [pltpu.VMEM((CHUNK, D_H), jnp.float32) for _ in range(NBUF)]
    + [pltpu.VMEM((CHUNK, D_H), jnp.float32) for _ in range(NBUF)]
    + [pltpu.SemaphoreType.DMA for _ in range(3 * NBUF)]
    + [pltpu.VMEM_SHARED((N_NODES, D_H), jnp.float32)],
    compiler_params=pltpu.CompilerParams(needs_layout_passes=False,
                                         use_tc_tiling_on_sc=False),
)
def _edge_kernel(xwf, qnf, knf, gsrcH, gqH, dstH, out, den_out,
                 qn_t, kn_t, eb_src, eb_gq, eb_dst, ex_v, den_t,
                 d0, d1, r0, r1, p0, p1,
                 sg0, sg1, sh0, sh1, ss0, ss1, acc_s):
    dst_v = [d0, d1]
    rows = [r0, r1]
    scb = [p0, p1]
    sg = [sg0, sg1]
    sh = [sh0, sh1]
    ss = [ss0, ss1]
    c = lax.axis_index("c")
    s = lax.axis_index("s")
    tid = c * NS + s

    base = jnp.where(c == 0, s * CH_C0, NS * CH_C0 + s * CH_C1)
    st = [
        pltpu.async_copy(qnf, qn_t, sg[0]),
        pltpu.async_copy(knf, kn_t, sg[1]),
        pltpu.async_copy(gsrcH.at[pl.ds(base, CH_MAX)], eb_src, sh[0]),
        pltpu.async_copy(gqH.at[pl.ds(base, CH_MAX)], eb_gq, sh[1]),
        pltpu.async_copy(dstH.at[pl.ds(base, CH_MAX)], eb_dst, ss[0]),
    ]
    for h in st:
        h.wait()

    zero16 = jnp.zeros((16,), jnp.float32)

    def zden_body(j, _):
        den_t[pl.ds(j * 16, 16)] = zero16
        return 0

    lax.fori_loop(0, N_NODES // 16, zden_body, 0)
    ex_v[pl.ds(CHUNK, 16)] = zero16

    def zbuf_body(j, _):
        scb[0][j, 0:16] = zero16
        scb[0][j, 16:32] = zero16
        return 0

    lax.fori_loop(0, CHUNK, zbuf_body, 0)

    nacc_full = N_NODES // 128
    nacc_rem = N_NODES - nacc_full * 128

    def zacc_body(i, _):
        j = s + i * NS

        @pl.when(j < nacc_full)
        def _():
            pltpu.sync_copy(scb[0], acc_s.at[pl.ds(j * 128, 128)])

        @pl.when(j == nacc_full)
        def _():
            pltpu.sync_copy(scb[0].at[pl.ds(0, nacc_rem)],
                            acc_s.at[pl.ds(nacc_full * 128, nacc_rem)])

        return 0

    lax.fori_loop(0, nacc_full // NS + 1, zacc_body, 0)
    plsc.subcore_barrier()

    for b in range(NBUF):
        pltpu.async_copy(xwf.at[eb_src.at[b, pl.ds(0, 64)]],
                         rows[b].at[pl.ds(0, 64)], sg[b])
        pltpu.async_copy(xwf.at[eb_src.at[b, pl.ds(64, 64)]],
                         rows[b].at[pl.ds(64, 64)], sh[b])

    ngroup = jnp.where(c == 0, CH_C0 // NBUF, CH_C1 // NBUF)

    def group_body(i, _):
        for b in range(NBUF):
            ch = i * NBUF + b
            pltpu.make_async_copy(xwf.at[eb_src.at[b, pl.ds(0, 64)]],
                                  rows[b].at[pl.ds(0, 64)], sg[b]).wait()
            pltpu.make_async_copy(xwf.at[eb_src.at[b, pl.ds(64, 64)]],
                                  rows[b].at[pl.ds(64, 64)], sh[b]).wait()

            @pl.when(i > 0)
            def _():
                pltpu.make_async_copy(scb[b], acc_s.at[dst_v[b]],
                                      ss[b]).wait()

            def icopy_body(k2, _, ch=ch, b=b):
                dst_v[b][pl.ds(k2 * 16, 16)] = eb_dst[ch, pl.ds(k2 * 16, 16)]
                return 0

            lax.fori_loop(0, CHUNK // 16, icopy_body, 0)

            def alpha_body(k2, _, ch=ch, b=b):
                idxq = eb_gq[ch, pl.ds(k2 * 16, 16)]
                idxs = eb_src[ch, pl.ds(k2 * 16, 16)]
                idxd = dst_v[b][pl.ds(k2 * 16, 16)]
                t = (plsc.load_gather(qn_t, [idxq])
                     + plsc.load_gather(kn_t, [idxs]))
                ex = jnp.exp(jnp.maximum(t, 0.2 * t))
                ex_v[pl.ds(k2 * 16, 16)] = ex
                plsc.addupdate_scatter(den_t, [idxd], ex)
                return 0

            lax.fori_loop(0, CHUNK // 16, alpha_body, 0)

            @plsc.parallel_loop(0, CHUNK, 1, unroll=8)
            def scale_body(e, b=b):
                exs = ex_v[pl.ds(e, 16)][0]
                scb[b][e, 0:16] = rows[b][e, 0:16] * exs
                scb[b][e, 16:32] = rows[b][e, 16:32] * exs

            @pl.when(i < ngroup - 1)
            def _():
                pltpu.async_copy(xwf.at[eb_src.at[ch + NBUF, pl.ds(0, 64)]],
                                 rows[b].at[pl.ds(0, 64)], sg[b])
                pltpu.async_copy(xwf.at[eb_src.at[ch + NBUF, pl.ds(64, 64)]],
                                 rows[b].at[pl.ds(64, 64)], sh[b])

            pltpu.async_copy(scb[b], acc_s.at[dst_v[b]], ss[b], add=True)
        return 0

    lax.fori_loop(0, ngroup, group_body, 0)

    for b in range(NBUF):
        pltpu.make_async_copy(scb[b], acc_s.at[dst_v[b]], ss[b]).wait()

    pltpu.sync_copy(den_t, den_out.at[tid])
    plsc.subcore_barrier()

    @pl.when(s == 0)
    def _():
        pltpu.sync_copy(acc_s, out.at[pl.ds(c * N_NODES, N_NODES)])


def kernel(x, edge_index, edge_type, W0, q0, k0, b0, g0, be0,
           W1, q1, k1, b1, g1, be1, W2, q2, k2, b2, g2, be2,
           mW1, mb1, mW2, mb2):
    f32 = jnp.float32
    zpad = jnp.zeros((EROWS_PAD - EROWS, 128), jnp.int32)
    src_p = jnp.concatenate([edge_index[0].reshape(EROWS, 128), zpad])
    dst_p = jnp.concatenate([edge_index[1].reshape(EROWS, 128), zpad])
    et_p = jnp.concatenate([edge_type.reshape(EROWS, 128), zpad])
    gsrc, gq, dstp = _prep(src_p, dst_p, et_p)

    def _tables(qn, kn):
        qnf = jnp.concatenate([qn.reshape(TQ), jnp.full((TQP - TQ,), -1e30, f32)])
        knf = jnp.concatenate([kn.reshape(TQ), jnp.zeros((TQP - TQ,), f32)])
        return qnf, knf

    def _edge(xw, qn, kn):
        qnf, knf = _tables(qn, kn)
        acc, den = _edge_kernel(xw.reshape(TQ, D_H), qnf, knf, gsrc, gq, dstp)
        return acc.reshape(NC, N_NODES, D_H), den

    xw, qn, kn = _proj0(x, W0, q0, k0)
    acc, den = _edge(xw, qn, kn)

    layer_params = [(b0, g0, be0, W1, q1, k1), (b1, g1, be1, W2, q2, k2)]
    for b, g, be, W, q, k in layer_params:
        xw, qn, kn = _finproj(acc, den, b.reshape(1, D_H), g.reshape(1, D_H),
                              be.reshape(1, D_H), W, q, k)
        acc, den = _edge(xw, qn, kn)

    return _head(acc, den, b2.reshape(1, D_H), g2.reshape(1, D_H),
                 be2.reshape(1, D_H), mW1, mb1.reshape(1, D_H), mW2,
                 mb2.reshape(1, 2))


# speedup vs baseline: 1.1579x; 1.0004x over previous
import functools

import jax
import jax.numpy as jnp
from jax import lax
from jax.experimental import pallas as pl
from jax.experimental.pallas import tpu as pltpu
from jax.experimental.pallas import tpu_sc as plsc

N_NODES = 10000
N_EDGES = 320000
D_IN = 128
D_H = 32
N_REL = 2

NC = 2
NS = 16
NTILES = NC * NS
CHUNK = 128
NBUF = 2
NCHUNK = 80
CH_C0 = 104
CH_C1 = 56
CH_MAX = 104
E_PAD = NCHUNK * CHUNK * NTILES
EROWS = N_EDGES // 128
EROWS_PAD = E_PAD // 128

TQ = 2 * N_NODES
SENT = TQ
TQP = TQ + 8


def _prep_body(src_ref, dst_ref, et_ref, gsrc_ref, gq_ref, dstp_ref):
    src = src_ref[...]
    dst = dst_ref[...]
    et = et_ref[...]
    rows = lax.broadcasted_iota(jnp.int32, (EROWS_PAD, 128), 0)
    valid = rows < EROWS
    gsrc_ref[...] = jnp.where(valid, et * N_NODES + src, 0)
    gq_ref[...] = jnp.where(valid, et * N_NODES + dst, SENT)
    dstp_ref[...] = jnp.where(valid, dst, 0)


_prep = pl.pallas_call(
    _prep_body,
    out_shape=(
        jax.ShapeDtypeStruct((EROWS_PAD, 128), jnp.int32),
        jax.ShapeDtypeStruct((EROWS_PAD, 128), jnp.int32),
        jax.ShapeDtypeStruct((EROWS_PAD, 128), jnp.int32),
    ),
)


def _project(h, W, q, k, xw_ref, qn_ref, kn_ref):
    for r in range(N_REL):
        xw = jnp.dot(h, W[r], preferred_element_type=jnp.float32)
        xw_ref[r] = xw
        qn_ref[r : r + 1, :] = lax.dot_general(
            q, xw, (((0,), (1,)), ((), ())), preferred_element_type=jnp.float32)
        kn_ref[r : r + 1, :] = lax.dot_general(
            k, xw, (((0,), (1,)), ((), ())), preferred_element_type=jnp.float32)


def _proj0_body(x_ref, W_ref, q_ref, k_ref, xw_ref, qn_ref, kn_ref):
    _project(x_ref[...], W_ref[...], q_ref[...], k_ref[...],
             xw_ref, qn_ref, kn_ref)


_proj0 = pl.pallas_call(
    _proj0_body,
    out_shape=(
        jax.ShapeDtypeStruct((N_REL, N_NODES, D_H), jnp.float32),
        jax.ShapeDtypeStruct((N_REL, N_NODES), jnp.float32),
        jax.ShapeDtypeStruct((N_REL, N_NODES), jnp.float32),
    ),
)


def _norm_h(acc_ref, den_ref, b_ref, g_ref, be_ref):
    sacc = acc_ref[0] + acc_ref[1]
    ones = jnp.ones((NTILES, 1), jnp.float32)
    den = lax.dot_general(den_ref[...], ones, (((0,), (0,)), ((), ())),
                          preferred_element_type=jnp.float32)
    h = sacc / (den + 1e-16) + b_ref[...]
    h = jnp.maximum(h, 0.0)
    mu = jnp.mean(h, axis=0, keepdims=True)
    var = jnp.mean((h - mu) * (h - mu), axis=0, keepdims=True)
    return (h - mu) * lax.rsqrt(var + 1e-5) * g_ref[...] + be_ref[...]


def _finproj_body(acc_ref, den_ref, b_ref, g_ref, be_ref, W_ref, q_ref, k_ref,
                  xw_ref, qn_ref, kn_ref):
    hn = _norm_h(acc_ref, den_ref, b_ref, g_ref, be_ref)
    _project(hn, W_ref[...], q_ref[...], k_ref[...], xw_ref, qn_ref, kn_ref)


_finproj = pl.pallas_call(
    _finproj_body,
    out_shape=(
        jax.ShapeDtypeStruct((N_REL, N_NODES, D_H), jnp.float32),
        jax.ShapeDtypeStruct((N_REL, N_NODES), jnp.float32),
        jax.ShapeDtypeStruct((N_REL, N_NODES), jnp.float32),
    ),
)


def _head_body(acc_ref, den_ref, b_ref, g_ref, be_ref, mW1_ref, mb1_ref,
               mW2_ref, mb2_ref, out_ref):
    hn = _norm_h(acc_ref, den_ref, b_ref, g_ref, be_ref)
    z = jnp.dot(hn, mW1_ref[...], preferred_element_type=jnp.float32)
    hh = jax.nn.sigmoid(z + mb1_ref[...])
    out_ref[...] = (jnp.dot(hh, mW2_ref[...], preferred_element_type=jnp.float32)
                    + mb2_ref[...])


_head = pl.pallas_call(
    _head_body,
    out_shape=jax.ShapeDtypeStruct((N_NODES, 2), jnp.float32),
)


_mesh = plsc.VectorSubcoreMesh(core_axis_name="c", subcore_axis_name="s")


@functools.partial(
    pl.kernel,
    out_type=(
        jax.ShapeDtypeStruct((NC * N_NODES, D_H), jnp.float32),
        jax.ShapeDtypeStruct((NTILES, N_NODES), jnp.float32),
    ),
    mesh=_mesh,
    scratch_types=[
        pltpu.VMEM((TQP,), jnp.float32),
        pltpu.VMEM((TQP,), jnp.float32),
        pltpu.VMEM((CH_MAX, 128), jnp.int32),
        pltpu.VMEM((CH_MAX, 128), jnp.int32),
        pltpu.VMEM((CH_MAX, 128), jnp.int32),
        pltpu.VMEM((CHUNK + 16,), jnp.float32),
        pltpu.VMEM((N_NODES,), jnp.float32),
    ]
    + [pltpu.VMEM((CHUNK,), jnp.int32) for _ in range(NBUF)]
    + [pltpu.VMEM((CHUNK, D_H), jnp.float32) for _ in range(NBUF)]
    + [pltpu.VMEM((CHUNK, D_H), jnp.float32) for _ in range(NBUF)]
    + [pltpu.SemaphoreType.DMA for _ in range(3 * NBUF)]
    + [pltpu.VMEM_SHARED((N_NODES, D_H), jnp.float32)],
    compiler_params=pltpu.CompilerParams(needs_layout_passes=False,
                                         use_tc_tiling_on_sc=False),
)
def _edge_kernel(xwf, qnf, knf, gsrcH, gqH, dstH, out, den_out,
                 qn_t, kn_t, eb_src, eb_gq, eb_dst, ex_v, den_t,
                 d0, d1, r0, r1, p0, p1,
                 sg0, sg1, sh0, sh1, ss0, ss1, acc_s):
    dst_v = [d0, d1]
    rows = [r0, r1]
    scb = [p0, p1]
    sg = [sg0, sg1]
    sh = [sh0, sh1]
    ss = [ss0, ss1]
    c = lax.axis_index("c")
    s = lax.axis_index("s")
    tid = c * NS + s

    base = jnp.where(c == 0, s * CH_C0, NS * CH_C0 + s * CH_C1)
    st = [
        pltpu.async_copy(qnf, qn_t, sg[0]),
        pltpu.async_copy(knf, kn_t, sg[1]),
        pltpu.async_copy(gsrcH.at[pl.ds(base, CH_MAX)], eb_src, sh[0]),
        pltpu.async_copy(gqH.at[pl.ds(base, CH_MAX)], eb_gq, sh[1]),
        pltpu.async_copy(dstH.at[pl.ds(base, CH_MAX)], eb_dst, ss[0]),
    ]
    for h in st:
        h.wait()

    zero16 = jnp.zeros((16,), jnp.float32)

    def zden_body(j, _):
        den_t[pl.ds(j * 16, 16)] = zero16
        return 0

    lax.fori_loop(0, N_NODES // 16, zden_body, 0)
    ex_v[pl.ds(CHUNK, 16)] = zero16

    def zbuf_body(j, _):
        scb[0][j, 0:16] = zero16
        scb[0][j, 16:32] = zero16
        return 0

    lax.fori_loop(0, CHUNK, zbuf_body, 0)

    nacc_full = N_NODES // 128
    nacc_rem = N_NODES - nacc_full * 128

    def zacc_body(i, _):
        j = s + i * NS

        @pl.when(j < nacc_full)
        def _():
            pltpu.sync_copy(scb[0], acc_s.at[pl.ds(j * 128, 128)])

        @pl.when(j == nacc_full)
        def _():
            pltpu.sync_copy(scb[0].at[pl.ds(0, nacc_rem)],
                            acc_s.at[pl.ds(nacc_full * 128, nacc_rem)])

        return 0

    lax.fori_loop(0, nacc_full // NS + 1, zacc_body, 0)
    plsc.subcore_barrier()

    for b in range(NBUF):
        pltpu.async_copy(xwf.at[eb_src.at[b, pl.ds(0, 64)]],
                         rows[b].at[pl.ds(0, 64)], sg[b])
        pltpu.async_copy(xwf.at[eb_src.at[b, pl.ds(64, 64)]],
                         rows[b].at[pl.ds(64, 64)], sh[b])

    ngroup = jnp.where(c == 0, CH_C0 // NBUF, CH_C1 // NBUF)

    def group_body(i, _):
        for b in range(NBUF):
            ch = i * NBUF + b
            pltpu.make_async_copy(xwf.at[eb_src.at[b, pl.ds(0, 64)]],
                                  rows[b].at[pl.ds(0, 64)], sg[b]).wait()
            pltpu.make_async_copy(xwf.at[eb_src.at[b, pl.ds(64, 64)]],
                                  rows[b].at[pl.ds(64, 64)], sh[b]).wait()

            @pl.when(i > 0)
            def _():
                pltpu.make_async_copy(scb[b], acc_s.at[dst_v[b]],
                                      ss[b]).wait()

            def icopy_body(k2, _, ch=ch, b=b):
                dst_v[b][pl.ds(k2 * 16, 16)] = eb_dst[ch, pl.ds(k2 * 16, 16)]
                return 0

            lax.fori_loop(0, CHUNK // 16, icopy_body, 0)

            def alpha_body(k2, _, ch=ch, b=b):
                idxq = eb_gq[ch, pl.ds(k2 * 16, 16)]
                idxs = eb_src[ch, pl.ds(k2 * 16, 16)]
                idxd = dst_v[b][pl.ds(k2 * 16, 16)]
                t = (plsc.load_gather(qn_t, [idxq])
                     + plsc.load_gather(kn_t, [idxs]))
                ex = jnp.exp(jnp.maximum(t, 0.2 * t))
                ex_v[pl.ds(k2 * 16, 16)] = ex
                plsc.addupdate_scatter(den_t, [idxd], ex)
                return 0

            lax.fori_loop(0, CHUNK // 16, alpha_body, 0)

            @plsc.parallel_loop(0, CHUNK, 1, unroll=8)
            def scale_body(e, b=b):
                exs = ex_v[pl.ds(e, 16)][0]
                scb[b][e, 0:16] = rows[b][e, 0:16] * exs
                scb[b][e, 16:32] = rows[b][e, 16:32] * exs

            @pl.when(i < ngroup - 1)
            def _():
                pltpu.async_copy(xwf.at[eb_src.at[ch + NBUF, pl.ds(0, 64)]],
                                 rows[b].at[pl.ds(0, 64)], sg[b])
                pltpu.async_copy(xwf.at[eb_src.at[ch + NBUF, pl.ds(64, 64)]],
                                 rows[b].at[pl.ds(64, 64)], sh[b])

            pltpu.async_copy(scb[b], acc_s.at[dst_v[b]], ss[b], add=True)
        return 0

    lax.fori_loop(0, ngroup, group_body, 0)

    for b in range(NBUF):
        pltpu.make_async_copy(scb[b], acc_s.at[dst_v[b]], ss[b]).wait()

    pltpu.sync_copy(den_t, den_out.at[tid])
    plsc.subcore_barrier()

    @pl.when(s == 0)
    def _():
        pltpu.sync_copy(acc_s, out.at[pl.ds(c * N_NODES, N_NODES)])


def kernel(x, edge_index, edge_type, W0, q0, k0, b0, g0, be0,
           W1, q1, k1, b1, g1, be1, W2, q2, k2, b2, g2, be2,
           mW1, mb1, mW2, mb2):
    f32 = jnp.float32
    zpad = jnp.zeros((EROWS_PAD - EROWS, 128), jnp.int32)
    src_p = jnp.concatenate([edge_index[0].reshape(EROWS, 128), zpad])
    dst_p = jnp.concatenate([edge_index[1].reshape(EROWS, 128), zpad])
    et_p = jnp.concatenate([edge_type.reshape(EROWS, 128), zpad])
    gsrc, gq, dstp = _prep(src_p, dst_p, et_p)

    def _tables(qn, kn):
        qnf = jnp.concatenate([qn.reshape(TQ), jnp.full((TQP - TQ,), -1e30, f32)])
        knf = jnp.concatenate([kn.reshape(TQ), jnp.zeros((TQP - TQ,), f32)])
        return qnf, knf

    def _edge(xw, qn, kn):
        qnf, knf = _tables(qn, kn)
        acc, den = _edge_kernel(xw.reshape(TQ, D_H), qnf, knf, gsrc, gq, dstp)
        return acc.reshape(NC, N_NODES, D_H), den

    xw, qn, kn = _proj0(x, W0, q0, k0)
    acc, den = _edge(xw, qn, kn)

    layer_params = [(b0, g0, be0, W1, q1, k1), (b1, g1, be1, W2, q2, k2)]
    for b, g, be, W, q, k in layer_params:
        xw, qn, kn = _finproj(acc, den, b.reshape(1, D_H), g.reshape(1, D_H),
                              be.reshape(1, D_H), W, q, k)
        acc, den = _edge(xw, qn, kn)

    return _head(acc, den, b2.reshape(1, D_H), g2.reshape(1, D_H),
                 be2.reshape(1, D_H), mW1, mb1.reshape(1, D_H), mW2,
                 mb2.reshape(1, 2))
